# baseline (speedup 1.0000x reference)
import jax
import jax.numpy as jnp
from jax import lax
from jax.experimental import pallas as pl
from jax.experimental.pallas import tpu as pltpu
from jax.experimental.pallas import tpu_sc as plsc
from jax.experimental.layout import Format, Layout

VOCAB = 1000000
EMBED_DIM = 64
MAX_SEQ_LEN = 200
SEQ_LEN = 200
BATCH = 4096

NC = 2
NS = 16
NW = NC * NS
L = 16
NV = EMBED_DIM // L

SEQ_PER_W = BATCH // NW
CHUNK = SEQ_LEN
N_CHUNKS = SEQ_PER_W
G_SPLIT = 128

_EPS = 1e-5
_INV_D = 1.0 / EMBED_DIM

_GDN = lax.GatherDimensionNumbers(
    offset_dims=(), collapsed_slice_dims=(0,), start_index_map=(0,))


def _permute(v, perm):
    return lax.gather(v, perm[:, None], _GDN, (1,),
                      mode=lax.GatherScatterMode.PROMISE_IN_BOUNDS)


def _allsum(v, perms):
    for p in perms:
        v = v + _permute(v, p)
    return v


def _rsqrt(a):
    i = plsc.bitcast(a, jnp.int32)
    i = 0x5F3759DF - lax.shift_right_logical(i, 1)
    y = plsc.bitcast(i, jnp.float32)
    half_a = 0.5 * a
    for _ in range(2):
        y = y * (1.5 - half_a * y * y)
    return y


def _body(x_hbm, table_hbm, pos_hbm, gamma_hbm, beta_hbm, out_hbm,
          idx_v, ib0, ib1, ob0, ob1, pos_v, gb_v,
          gsem0, gsem1, wsem0, wsem1):
    wid = lax.axis_index("s") * NC + lax.axis_index("c")
    base_seq = wid * SEQ_PER_W

    pltpu.sync_copy(x_hbm.at[pl.ds(base_seq, SEQ_PER_W)], idx_v)
    pltpu.sync_copy(pos_hbm.at[0], pos_v)
    pltpu.sync_copy(gamma_hbm, gb_v.at[0])
    pltpu.sync_copy(beta_hbm, gb_v.at[1])

    g = [gb_v[0, pl.ds(i * L, L)] for i in range(NV)]
    b = [gb_v[1, pl.ds(i * L, L)] for i in range(NV)]
    lanes = lax.iota(jnp.int32, L)
    perms = [lax.bitwise_xor(lanes, jnp.int32(1 << i)) for i in range(4)]

    def fire_gather(k, ibuf, gsem):
        pltpu.async_copy(table_hbm.at[idx_v.at[k, pl.ds(0, G_SPLIT)]],
                         ibuf.at[pl.ds(0, G_SPLIT)], gsem)
        pltpu.async_copy(table_hbm.at[idx_v.at[k, pl.ds(G_SPLIT,
                                                        CHUNK - G_SPLIT)]],
                         ibuf.at[pl.ds(G_SPLIT, CHUNK - G_SPLIT)], gsem)

    def wait_gather(ibuf, gsem):
        pltpu.make_async_copy(out_hbm.at[0], ibuf, gsem).wait()

    def fire_write(k, obuf, wsem):
        pltpu.async_copy(obuf, out_hbm.at[base_seq + k], wsem)

    def wait_write(obuf, wsem):
        pltpu.make_async_copy(obuf, out_hbm.at[0], wsem).wait()

    def compute(ibuf, obuf):
        def row(r):
            h = [ibuf[r, pl.ds(i * L, L)] + pos_v[r, pl.ds(i * L, L)]
                 for i in range(NV)]
            tot = (h[0] + h[1]) + (h[2] + h[3])
            sq = (h[0] * h[0] + h[1] * h[1]) + (h[2] * h[2] + h[3] * h[3])
            mean = _allsum(tot, perms) * _INV_D
            var = _allsum(sq, perms) * _INV_D - mean * mean
            rstd = _rsqrt(var + _EPS)
            off = mean * rstd
            for i in range(NV):
                obuf[r, pl.ds(i * L, L)] = (h[i] * rstd - off) * g[i] + b[i]

        plsc.parallel_loop(0, CHUNK, unroll=4)(row)

    fire_gather(0, ib0, gsem0)
    fire_gather(1, ib1, gsem1)

    wait_gather(ib0, gsem0)
    compute(ib0, ob0)
    fire_write(0, ob0, wsem0)
    fire_gather(2, ib0, gsem0)
    wait_gather(ib1, gsem1)
    compute(ib1, ob1)
    fire_write(1, ob1, wsem1)
    fire_gather(3, ib1, gsem1)

    def pair_body(j, carry):
        k0 = 2 * j + 2
        wait_gather(ib0, gsem0)
        wait_write(ob0, wsem0)
        compute(ib0, ob0)
        fire_write(k0, ob0, wsem0)
        fire_gather(k0 + 2, ib0, gsem0)
        wait_gather(ib1, gsem1)
        wait_write(ob1, wsem1)
        compute(ib1, ob1)
        fire_write(k0 + 1, ob1, wsem1)
        fire_gather(k0 + 3, ib1, gsem1)
        return carry

    lax.fori_loop(0, (N_CHUNKS - 4) // 2, pair_body, 0)

    wait_gather(ib0, gsem0)
    wait_write(ob0, wsem0)
    compute(ib0, ob0)
    fire_write(N_CHUNKS - 2, ob0, wsem0)
    wait_gather(ib1, gsem1)
    wait_write(ob1, wsem1)
    compute(ib1, ob1)
    fire_write(N_CHUNKS - 1, ob1, wsem1)

    wait_write(ob0, wsem0)
    wait_write(ob1, wsem1)


@jax.jit
def _run(x, table, pos, gamma, beta):
    mesh = plsc.VectorSubcoreMesh(core_axis_name="c", subcore_axis_name="s",
                                  num_cores=NC, num_subcores=NS)
    f = pl.kernel(
        _body,
        out_type=jax.ShapeDtypeStruct((BATCH, SEQ_LEN, EMBED_DIM),
                                      jnp.float32),
        mesh=mesh,
        compiler_params=pltpu.CompilerParams(needs_layout_passes=False,
                                             use_tc_tiling_on_sc=False),
        scratch_types=[
            pltpu.VMEM((SEQ_PER_W, SEQ_LEN), jnp.int32),
            pltpu.VMEM((CHUNK, EMBED_DIM), jnp.float32),
            pltpu.VMEM((CHUNK, EMBED_DIM), jnp.float32),
            pltpu.VMEM((CHUNK, EMBED_DIM), jnp.float32),
            pltpu.VMEM((CHUNK, EMBED_DIM), jnp.float32),
            pltpu.VMEM((SEQ_LEN, EMBED_DIM), jnp.float32),
            pltpu.VMEM((2, EMBED_DIM), jnp.float32),
            pltpu.SemaphoreType.DMA,
            pltpu.SemaphoreType.DMA,
            pltpu.SemaphoreType.DMA,
            pltpu.SemaphoreType.DMA,
        ],
    )
    return f(x, table, pos, gamma, beta)


_pinned = {}


def _pinned_run(dev):
    fn = _pinned.get(dev)
    if fn is None:
        fn = jax.jit(
            _run,
            out_shardings=Format(
                Layout(major_to_minor=(0, 1, 2), tiling=((8,),)),
                jax.sharding.SingleDeviceSharding(dev)))
        _pinned[dev] = fn
    return fn


def kernel(x, token_table, pos_embed, gamma, beta):
    try:
        dev = list(x.devices())[0]
    except Exception:
        return _run(x, token_table, pos_embed, gamma, beta)
    return _pinned_run(dev)(x, token_table, pos_embed, gamma, beta)
